# baseline (speedup 1.0000x reference)
import functools

import jax
import jax.numpy as jnp
from jax import lax
from jax.experimental import pallas as pl
from jax.experimental.pallas import tpu as pltpu
from jax.experimental.pallas import tpu_sc as plsc

B = 4096
S = 200
D = 32
V = 1000000

_info = plsc.get_sparse_core_info()
NC = _info.num_cores
NS = _info.num_subcores
NW = NC * NS

BT = B // 128
ST = S // 8
DT = D // 8
NBUF = 4

TCB = 4096
TG = (V + TCB - 1) // TCB
VP = TG * TCB

assert BT == NW and S % NBUF == 0


def _tc_body(i_ref, o_ref):
  r = lax.broadcasted_iota(jnp.int32, (D, D), 0)
  c = lax.broadcasted_iota(jnp.int32, (D, D), 1)
  eye = (r == c).astype(jnp.float32)
  a = lax.dot_general(i_ref[...], eye, (((0,), (0,)), ((), ())),
                      precision=lax.Precision.HIGHEST,
                      preferred_element_type=jnp.float32)
  for m in range(TCB // 512):
    for k in range(4):
      lo = 512 * m + 128 * k
      o_ref[128 * m:128 * (m + 1), 32 * k:32 * (k + 1)] = a[lo:lo + 128, :]


_tc_transpose = pl.pallas_call(
    _tc_body,
    grid=(TG,),
    in_specs=[pl.BlockSpec((D, TCB), lambda g: (0, g))],
    out_specs=pl.BlockSpec((TCB // 4, 128), lambda g: (g, 0)),
    out_shape=jax.ShapeDtypeStruct((TG * (TCB // 4), 128), jnp.float32),
)


def _sc_body(xv_hbm, tok_hbm, pos_hbm, out_hbm,
             idx_v, pos_v, rows, tbuf, gsems, osems, lsem):
  cid = lax.axis_index("c")
  sid = lax.axis_index("s")
  wid = sid * NC + cid

  for a in range(ST):
    pltpu.async_copy(xv_hbm.at[a, wid], idx_v.at[a], lsem)
  pltpu.async_copy(pos_hbm, pos_v, lsem)
  for a in range(ST):
    pltpu.make_async_copy(xv_hbm.at[a, wid], idx_v.at[a], lsem).wait()
  pltpu.make_async_copy(pos_hbm, pos_v, lsem).wait()

  def permbody(p, carry):
    a = p // 8
    r = p % 8
    for c in range(8):
      v = idx_v[a, r, pl.ds(16 * c, 16)]
      idx_v[a, r, pl.ds(16 * c, 16)] = (
          (v & -512) + ((v & 127) << 2) + ((v >> 7) & 3))
    return carry

  lax.fori_loop(0, ST * 8, permbody, 0)

  def fire_gather(s, b):
    a = s // 8
    r = s % 8
    pltpu.async_copy(tok_hbm.at[idx_v.at[a, r]], rows[b], gsems[b])

  def drain_gather(s, b):
    a = s // 8
    r = s % 8
    pltpu.make_async_copy(tok_hbm.at[idx_v.at[a, r]], rows[b],
                          gsems[b]).wait()

  def compute(s, b):
    def rbody(rr, carry):
      lane = lax.iota(jnp.int32, 16)
      dt_lo = lane // 8
      d8_lo = lane % 8
      dt_hi = dt_lo + 2
      pv0 = pos_v[s, pl.ds(0, 16)]
      pv1 = pos_v[s, pl.ds(16, 16)]
      for k in range(8):
        r = rr * 8 + k
        rsp = jnp.full((16,), r, dtype=jnp.int32)
        v0 = rows[b][r, pl.ds(0, 16)] + pv0
        v1 = rows[b][r, pl.ds(16, 16)] + pv1
        plsc.store_scatter(tbuf[b], [dt_lo, d8_lo, rsp], v0)
        plsc.store_scatter(tbuf[b], [dt_hi, d8_lo, rsp], v1)
      return carry

    lax.fori_loop(0, 16, rbody, 0)

  def fire_out(s, b):
    for dt in range(DT):
      pltpu.async_copy(tbuf[b].at[dt, slice(None), pl.ds(0, 128)],
                       out_hbm.at[s, dt, wid], osems[b])

  def wait_out(s, b):
    for dt in range(DT):
      pltpu.make_async_copy(tbuf[b].at[dt, slice(None), pl.ds(0, 128)],
                            out_hbm.at[s, dt, wid], osems[b]).wait()

  for b in range(NBUF):
    fire_gather(b, b)

  def sbody(ss, carry):
    for b in range(NBUF):
      s = NBUF * ss + b

      @pl.when(ss > 0)
      def _wait_prev():
        wait_out(s - NBUF, b)

      drain_gather(s, b)
      compute(s, b)
      fire_out(s, b)

      @pl.when(ss < S // NBUF - 1)
      def _refill():
        fire_gather(s + NBUF, b)
    return carry

  lax.fori_loop(0, S // NBUF, sbody, 0)

  for b in range(NBUF):
    wait_out(S - NBUF + b, b)


def _make_sc_embed():
  scratch = [
      pltpu.VMEM((ST, 8, 128), jnp.int32),
      pltpu.VMEM((S, D), jnp.float32),
      [pltpu.VMEM((128, D), jnp.float32) for _ in range(NBUF)],
      [pltpu.VMEM((DT, 8, 129), jnp.float32) for _ in range(NBUF)],
      [pltpu.SemaphoreType.DMA for _ in range(NBUF)],
      [pltpu.SemaphoreType.DMA for _ in range(NBUF)],
      pltpu.SemaphoreType.DMA,
  ]
  return pl.kernel(
      _sc_body,
      out_type=jax.ShapeDtypeStruct((S, DT, BT, 8, 128), jnp.float32),
      mesh=plsc.VectorSubcoreMesh(core_axis_name="c", subcore_axis_name="s"),
      compiler_params=pltpu.CompilerParams(use_tc_tiling_on_sc=False,
                                           needs_layout_passes=False),
      scratch_types=scratch,
  )


_sc_embed = _make_sc_embed()


@jax.jit
def kernel(x, token_table, pos_table):
  xv = x.astype(jnp.int32).T.reshape(ST, 8, BT, 128).transpose(0, 2, 1, 3)
  tok_rows = _tc_transpose(token_table.T).reshape(VP, D)
  p = _sc_embed(xv, tok_rows, pos_table)
  return p.transpose(2, 4, 0, 1, 3).reshape(B, S, D)


# speedup vs baseline: 1.5820x; 1.5820x over previous
import jax
import jax.numpy as jnp
from jax import lax
from jax.experimental import pallas as pl
from jax.experimental.pallas import tpu as pltpu
from jax.experimental.pallas import tpu_sc as plsc

B = 4096
S = 200
D = 32
V = 1000000

_info = plsc.get_sparse_core_info()
NC = _info.num_cores
NS = _info.num_subcores
NW = NC * NS

BT = B // 128
ST = S // 8
DT = D // 8
NBUF = 4

TCB = 32768
TG = (V + TCB - 1) // TCB
VP = TG * TCB

assert BT == NW and S % NBUF == 0


def _tc_body(i_ref, o_ref):
  r = lax.broadcasted_iota(jnp.int32, (D, D), 0)
  c = lax.broadcasted_iota(jnp.int32, (D, D), 1)
  eye = (r == c).astype(jnp.float32)
  a = lax.dot_general(i_ref[...], eye, (((0,), (0,)), ((), ())),
                      preferred_element_type=jnp.float32)
  for m in range(TCB // 512):
    for k in range(4):
      lo = 512 * m + 128 * k
      o_ref[128 * m:128 * (m + 1), 32 * k:32 * (k + 1)] = a[lo:lo + 128, :]


_tc_transpose = pl.pallas_call(
    _tc_body,
    grid=(TG,),
    in_specs=[pl.BlockSpec((D, TCB), lambda g: (0, g))],
    out_specs=pl.BlockSpec((TCB // 4, 128), lambda g: (g, 0)),
    out_shape=jax.ShapeDtypeStruct((TG * (TCB // 4), 128), jnp.float32),
)


def _sc_body(xv_hbm, tok_hbm, pos_hbm, out_hbm,
             idx_v, pos_v, rows, tbuf, gsems, osems, lsem):
  cid = lax.axis_index("c")
  sid = lax.axis_index("s")
  wid = sid * NC + cid

  for a in range(ST):
    pltpu.async_copy(xv_hbm.at[a, wid], idx_v.at[a], lsem)
  pltpu.async_copy(pos_hbm, pos_v, lsem)
  for a in range(ST):
    pltpu.make_async_copy(xv_hbm.at[a, wid], idx_v.at[a], lsem).wait()
  pltpu.make_async_copy(pos_hbm, pos_v, lsem).wait()

  def permbody(p, carry):
    a = p // 8
    r = p % 8
    for c in range(8):
      v = idx_v[a, r, pl.ds(16 * c, 16)]
      idx_v[a, r, pl.ds(16 * c, 16)] = (
          (v & -512) + ((v & 127) << 2) + ((v >> 7) & 3))
    return carry

  lax.fori_loop(0, ST * 8, permbody, 0)

  def fire_gather(s, b):
    a = s // 8
    r = s % 8
    pltpu.async_copy(tok_hbm.at[idx_v.at[a, r]], rows[b], gsems[b])

  def drain_gather(s, b):
    a = s // 8
    r = s % 8
    pltpu.make_async_copy(tok_hbm.at[idx_v.at[a, r]], rows[b],
                          gsems[b]).wait()

  def compute(s, b):
    def rbody(rr, carry):
      lane = lax.iota(jnp.int32, 16)
      dt_lo = lane // 8
      d8_lo = lane % 8
      dt_hi = dt_lo + 2
      pv0 = pos_v[s, pl.ds(0, 16)]
      pv1 = pos_v[s, pl.ds(16, 16)]
      for k in range(8):
        r = rr * 8 + k
        rsp = jnp.full((16,), r, dtype=jnp.int32)
        v0 = rows[b][r, pl.ds(0, 16)] + pv0
        v1 = rows[b][r, pl.ds(16, 16)] + pv1
        plsc.store_scatter(tbuf[b], [dt_lo, d8_lo, rsp], v0)
        plsc.store_scatter(tbuf[b], [dt_hi, d8_lo, rsp], v1)
      return carry

    lax.fori_loop(0, 16, rbody, 0)

  def fire_out(s, b):
    pltpu.async_copy(tbuf[b].at[slice(None), slice(None), pl.ds(0, 128)],
                     out_hbm.at[s, slice(None), wid], osems[b])

  def wait_out(s, b):
    pltpu.make_async_copy(tbuf[b].at[slice(None), slice(None), pl.ds(0, 128)],
                          out_hbm.at[s, slice(None), wid], osems[b]).wait()

  for b in range(NBUF):
    fire_gather(b, b)

  def sbody(ss, carry):
    for b in range(NBUF):
      s = NBUF * ss + b

      @pl.when(ss > 0)
      def _wait_prev():
        wait_out(s - NBUF, b)

      drain_gather(s, b)
      compute(s, b)
      fire_out(s, b)

      @pl.when(ss < S // NBUF - 1)
      def _refill():
        fire_gather(s + NBUF, b)
    return carry

  lax.fori_loop(0, S // NBUF, sbody, 0)

  for b in range(NBUF):
    wait_out(S - NBUF + b, b)


def _make_sc_embed():
  scratch = [
      pltpu.VMEM((ST, 8, 128), jnp.int32),
      pltpu.VMEM((S, D), jnp.float32),
      [pltpu.VMEM((128, D), jnp.float32) for _ in range(NBUF)],
      [pltpu.VMEM((DT, 8, 129), jnp.float32) for _ in range(NBUF)],
      [pltpu.SemaphoreType.DMA for _ in range(NBUF)],
      [pltpu.SemaphoreType.DMA for _ in range(NBUF)],
      pltpu.SemaphoreType.DMA,
  ]
  return pl.kernel(
      _sc_body,
      out_type=jax.ShapeDtypeStruct((S, DT, BT, 8, 128), jnp.float32),
      mesh=plsc.VectorSubcoreMesh(core_axis_name="c", subcore_axis_name="s"),
      compiler_params=pltpu.CompilerParams(use_tc_tiling_on_sc=False,
                                           needs_layout_passes=False),
      scratch_types=scratch,
  )


_sc_embed = _make_sc_embed()


@jax.jit
def kernel(x, token_table, pos_table):
  xv = x.astype(jnp.int32).T.reshape(ST, 8, BT, 128).transpose(0, 2, 1, 3)
  tok_rows = _tc_transpose(token_table.T).reshape(VP, D)
  p = _sc_embed(xv, tok_rows, pos_table)
  return p.transpose(2, 4, 0, 1, 3).reshape(B, S, D)
